# Initial kernel scaffold; baseline (speedup 1.0000x reference)
import jax
import jax.numpy as jnp
from jax.experimental import pallas as pl


def kernel(x_raw, pe_nodefeat, edge_index, batch_vec, W1, b1, Wmu, bmu, Wlv, blv, pool_w, eps):
    raise NotImplementedError("write your pallas kernel here")


# speedup vs baseline: 1.2742x; 1.2742x over previous
import functools

import jax
import jax.numpy as jnp
import numpy as np
from jax.experimental import pallas as pl
from jax.experimental.pallas import tpu as pltpu

_N = 10000
_E = 320000
_K = 5000


def _mm1_body(x_ref, pe_ref, w_ref, o_ref):
    x = jnp.concatenate([x_ref[...], pe_ref[...]], axis=-1)
    o_ref[...] = jax.lax.dot_general(
        x, w_ref[...], (((1,), (0,)), ((), ())),
        preferred_element_type=jnp.float32)


def _xw1(x_raw, pe, W1):
    blk = 1000
    return pl.pallas_call(
        _mm1_body,
        grid=(_N // blk,),
        in_specs=[
            pl.BlockSpec((blk, 128), lambda i: (i, 0)),
            pl.BlockSpec((blk, 64), lambda i: (i, 0)),
            pl.BlockSpec((192, 128), lambda i: (0, 0)),
        ],
        out_specs=pl.BlockSpec((blk, 128), lambda i: (i, 0)),
        out_shape=jax.ShapeDtypeStruct((_N, 128), jnp.float32),
    )(x_raw, pe, W1)


def _mm2_body(h_ref, w_ref, o_ref):
    o_ref[...] = jax.lax.dot_general(
        h_ref[...], w_ref[...], (((1,), (0,)), ((), ())),
        preferred_element_type=jnp.float32)


def _mm2(h, W):
    blk = 1000
    return pl.pallas_call(
        _mm2_body,
        grid=(_N // blk,),
        in_specs=[
            pl.BlockSpec((blk, 128), lambda i: (i, 0)),
            pl.BlockSpec((128, 128), lambda i: (0, 0)),
        ],
        out_specs=pl.BlockSpec((blk, 128), lambda i: (i, 0)),
        out_shape=jax.ShapeDtypeStruct((_N, 128), jnp.float32),
    )(h, W)


def kernel(x_raw, pe_nodefeat, edge_index, batch_vec, W1, b1, Wmu, bmu, Wlv,
           blv, pool_w, eps):
    src = edge_index[0]
    dst = edge_index[1]

    deg = jnp.zeros((_N,), jnp.float32).at[dst].add(1.0) + 1.0
    dinv = 1.0 / jnp.sqrt(deg)
    coef = dinv[src] * dinv[dst]
    self_coef = (dinv * dinv)[:, None]

    xw = _xw1(x_raw, pe_nodefeat, W1)
    agg = jnp.zeros_like(xw).at[dst].add(xw[src] * coef[:, None])
    h = jax.nn.relu(agg + xw * self_coef + b1)

    Wcat = jnp.concatenate([Wmu, Wlv], axis=1)
    bcat = jnp.concatenate([bmu, blv])
    xw2 = _mm2(h, Wcat)
    agg2 = jnp.zeros_like(xw2).at[dst].add(xw2[src] * coef[:, None])
    out2 = agg2 + xw2 * self_coef + bcat
    mu = out2[:, :64]
    logvar = out2[:, 64:]

    std = jnp.exp(0.5 * logvar)
    z = mu + eps * std
    score = z @ pool_w / (jnp.linalg.norm(pool_w) + 1e-16)
    top_scores, perm = jax.lax.top_k(score, _K)
    gate = jnp.tanh(top_scores)
    z_pool = z[perm] * gate[:, None]
    mu_pool = mu[perm]
    logvar_pool = logvar[perm]
    return (z_pool, mu_pool, logvar_pool)


# speedup vs baseline: 1.5862x; 1.2449x over previous
import functools

import jax
import jax.numpy as jnp
from jax import lax
from jax.experimental import pallas as pl
from jax.experimental.pallas import tpu as pltpu
from jax.experimental.pallas import tpu_sc as plsc

_N = 10000
_E = 320000
_K = 5000
_PN = 10240
_NC, _NS = 2, 16
_NW = _NC * _NS
_EPT = _E // _NW
_EB = 128
_NBLK = _E // _EB
_BPT = 80

_mesh = plsc.VectorSubcoreMesh(core_axis_name="c", subcore_axis_name="s")
_sc_params = pltpu.CompilerParams(needs_layout_passes=False)


@functools.partial(
    pl.kernel,
    out_type=jax.ShapeDtypeStruct((_NW, 1, _PN), jnp.float32),
    mesh=_mesh,
    compiler_params=_sc_params,
    scratch_types=[
        pltpu.VMEM((_EPT,), jnp.int32),
        pltpu.VMEM((1, _PN), jnp.float32),
    ],
)
def _deg_sc(dst_hbm, out_hbm, dbuf, hist):
    c = lax.axis_index("c")
    s = lax.axis_index("s")
    wid = s * _NC + c

    @pl.loop(0, _PN // 16)
    def _zero(i):
        hist[0, pl.ds(i * 16, 16)] = jnp.zeros((16,), jnp.float32)

    pltpu.sync_copy(dst_hbm.at[pl.ds(wid * _EPT, _EPT)], dbuf)

    ones = jnp.ones((16,), jnp.float32)
    zeros_i = jnp.zeros((16,), jnp.int32)

    @pl.loop(0, _EPT // 16)
    def _histo(i):
        d16 = dbuf[pl.ds(i * 16, 16)]
        plsc.addupdate_scatter(hist, [zeros_i, d16], ones)

    pltpu.sync_copy(hist, out_hbm.at[wid])


@functools.partial(
    pl.kernel,
    out_type=jax.ShapeDtypeStruct((_E, 128), jnp.float32),
    mesh=_mesh,
    compiler_params=_sc_params,
    scratch_types=[
        pltpu.VMEM((_EB,), jnp.int32),
        pltpu.VMEM((_EB,), jnp.int32),
        pltpu.VMEM((_EB,), jnp.float32),
        pltpu.VMEM((_EB,), jnp.float32),
        pltpu.VMEM((_EB, 128), jnp.float32),
        pltpu.VMEM((_EB, 128), jnp.float32),
        pltpu.SemaphoreType.DMA,
        pltpu.SemaphoreType.DMA,
    ],
)
def _gm_sc(xw_hbm, src_hbm, coef_hbm, out_hbm,
           s0, s1, c0, c1, r0, r1, g0, g1):
    c = lax.axis_index("c")
    s = lax.axis_index("s")
    wid = s * _NC + c
    sbufs, cbufs, rbufs, sems = (s0, s1), (c0, c1), (r0, r1), (g0, g1)

    def _issue(k, p):
        g = k * _NW + wid

        @pl.when(g < _NBLK)
        def _():
            pltpu.sync_copy(src_hbm.at[pl.ds(g * _EB, _EB)], sbufs[p])
            pltpu.sync_copy(coef_hbm.at[pl.ds(g * _EB, _EB)], cbufs[p])
            pltpu.make_async_copy(
                xw_hbm.at[sbufs[p]], rbufs[p], sems[p]).start()

    def _drain(k, p):
        g = k * _NW + wid

        @pl.when(g < _NBLK)
        def _():
            pltpu.make_async_copy(
                xw_hbm.at[sbufs[p]], rbufs[p], sems[p]).wait()

            @pl.loop(0, _EB // 16)
            def _scale(j):
                cv = cbufs[p][pl.ds(j * 16, 16)]
                for jj in range(16):
                    e = j * 16 + jj
                    bc = jnp.full((16,), cv[jj], jnp.float32)
                    for kk in range(8):
                        rbufs[p][e, pl.ds(kk * 16, 16)] = (
                            rbufs[p][e, pl.ds(kk * 16, 16)] * bc)

            pltpu.sync_copy(rbufs[p], out_hbm.at[pl.ds(g * _EB, _EB)])

    _issue(0, 0)

    @pl.loop(0, _BPT // 2)
    def _edge_loop(k):
        idx0 = k * 2
        _issue(idx0 + 1, 1)
        _drain(idx0, 0)
        _issue(idx0 + 2, 0)
        _drain(idx0 + 1, 1)


def _tc1_body(x_ref, pe_ref, w_ref, o_ref):
    x = jnp.concatenate([x_ref[...], pe_ref[...]], axis=-1)
    o_ref[...] = lax.dot_general(x, w_ref[...], (((1,), (0,)), ((), ())),
                                 preferred_element_type=jnp.float32)


def _tc1(x_raw, pe, W1):
    blk = 1280
    return pl.pallas_call(
        _tc1_body,
        grid=(_PN // blk,),
        in_specs=[
            pl.BlockSpec((blk, 128), lambda i: (i, 0)),
            pl.BlockSpec((blk, 64), lambda i: (i, 0)),
            pl.BlockSpec((192, 128), lambda i: (0, 0)),
        ],
        out_specs=pl.BlockSpec((blk, 128), lambda i: (i, 0)),
        out_shape=jax.ShapeDtypeStruct((_PN, 128), jnp.float32),
    )(x_raw, pe, W1)


def _tc2_body(a_ref, xw_ref, sw_ref, b_ref, w_ref, o_ref):
    agg = a_ref[...] + xw_ref[...] * sw_ref[...]
    h = jnp.maximum(agg + b_ref[...], 0.0)
    o_ref[...] = lax.dot_general(h, w_ref[...], (((1,), (0,)), ((), ())),
                                 preferred_element_type=jnp.float32)


def _tc2(agg1, xw1, selfw, b1_2d, Wcat):
    blk = 1280
    return pl.pallas_call(
        _tc2_body,
        grid=(_PN // blk,),
        in_specs=[
            pl.BlockSpec((blk, 128), lambda i: (i, 0)),
            pl.BlockSpec((blk, 128), lambda i: (i, 0)),
            pl.BlockSpec((blk, 1), lambda i: (i, 0)),
            pl.BlockSpec((1, 128), lambda i: (0, 0)),
            pl.BlockSpec((128, 128), lambda i: (0, 0)),
        ],
        out_specs=pl.BlockSpec((blk, 128), lambda i: (i, 0)),
        out_shape=jax.ShapeDtypeStruct((_PN, 128), jnp.float32),
    )(agg1, xw1, selfw, b1_2d, Wcat)


def _tc3_body(a_ref, xw_ref, sw_ref, b_ref, mu_ref, lv_ref):
    agg = a_ref[...] + xw_ref[...] * sw_ref[...]
    out2 = agg + b_ref[...]
    mu_ref[...] = out2[:, :64]
    lv_ref[...] = out2[:, 64:]


def _tc3(agg2, xw2, selfw, bcat_2d):
    blk = 1280
    o64 = jax.ShapeDtypeStruct((_PN, 64), jnp.float32)
    return pl.pallas_call(
        _tc3_body,
        grid=(_PN // blk,),
        in_specs=[
            pl.BlockSpec((blk, 128), lambda i: (i, 0)),
            pl.BlockSpec((blk, 128), lambda i: (i, 0)),
            pl.BlockSpec((blk, 1), lambda i: (i, 0)),
            pl.BlockSpec((1, 128), lambda i: (0, 0)),
        ],
        out_specs=[
            pl.BlockSpec((blk, 64), lambda i: (i, 0)),
            pl.BlockSpec((blk, 64), lambda i: (i, 0)),
        ],
        out_shape=[o64, o64],
    )(agg2, xw2, selfw, bcat_2d)


def kernel(x_raw, pe_nodefeat, edge_index, batch_vec, W1, b1, Wmu, bmu, Wlv,
           blv, pool_w, eps):
    src = edge_index[0]
    dst = edge_index[1]
    pad = _PN - _N
    x_p = jnp.pad(x_raw, ((0, pad), (0, 0)))
    pe_p = jnp.pad(pe_nodefeat, ((0, pad), (0, 0)))

    hists = _deg_sc(dst)
    deg = jnp.sum(hists, axis=(0, 1)) + 1.0
    dinv = 1.0 / jnp.sqrt(deg)
    coef = dinv[src] * dinv[dst]
    selfw = (dinv * dinv).reshape(_PN, 1)

    xw1 = _tc1(x_p, pe_p, W1)
    upd1 = _gm_sc(xw1, src, coef)
    agg1 = jnp.zeros((_PN, 128), jnp.float32).at[dst].add(upd1)

    Wcat = jnp.concatenate([Wmu, Wlv], axis=1)
    bcat = jnp.concatenate([bmu, blv]).reshape(1, 128)
    xw2 = _tc2(agg1, xw1, selfw, b1.reshape(1, 128), Wcat)
    upd2 = _gm_sc(xw2, src, coef)
    agg2 = jnp.zeros((_PN, 128), jnp.float32).at[dst].add(upd2)

    mu, logvar = _tc3(agg2, xw2, selfw, bcat)
    mu, logvar = mu[:_N], logvar[:_N]

    std = jnp.exp(0.5 * logvar)
    z = mu + eps * std
    score = z @ pool_w / (jnp.linalg.norm(pool_w) + 1e-16)
    top_scores, perm = jax.lax.top_k(score, _K)
    gate = jnp.tanh(top_scores)
    z_pool = z[perm] * gate[:, None]
    mu_pool = mu[perm]
    logvar_pool = logvar[perm]
    return (z_pool, mu_pool, logvar_pool)
